# Initial kernel scaffold; baseline (speedup 1.0000x reference)
import jax
import jax.numpy as jnp
from jax.experimental import pallas as pl


def kernel(raw, anchors, img_size):
    raise NotImplementedError("write your pallas kernel here")


# speedup vs baseline: 2.6488x; 2.6488x over previous
import jax
import jax.numpy as jnp
from jax.experimental import pallas as pl
from jax.experimental.pallas import tpu as pltpu


def _decode_body(scal_ref, x_ref, o_ref):
    i = pl.program_id(0)
    a = i % 3
    nG = 76
    L = nG * nG
    x = x_ref[0]
    c = jax.lax.broadcasted_iota(jnp.int32, (85, L), 0)
    g = jax.lax.broadcasted_iota(jnp.int32, (85, L), 1)
    is_wh = jnp.logical_or(c == 2, c == 3)
    e = jnp.exp(jnp.where(is_wh, x, -x))
    s = 1.0 / (1.0 + e)
    stride = scal_ref[0]
    aw = scal_ref[1 + 2 * a]
    ah = scal_ref[2 + 2 * a]
    mx = (g % nG).astype(jnp.float32)
    my = (g // nG).astype(jnp.float32)
    anch = jnp.where(c == 2, aw, ah)
    res = jnp.where(c == 0, (s + mx) * stride,
          jnp.where(c == 1, (s + my) * stride,
          jnp.where(is_wh, e * anch, s)))
    o_ref[0] = res.T


def kernel(raw, anchors, img_size):
    nB, nCHA, nG, _ = raw.shape
    nA = anchors.shape[0]
    nCH = nCHA // nA
    L = nG * nG
    stride = (img_size // nG).astype(jnp.float32) if hasattr(img_size, "astype") \
        else jnp.float32(img_size // nG)
    scal = jnp.concatenate([jnp.reshape(stride, (1,)),
                            anchors.astype(jnp.float32).reshape(-1)])
    x = raw.reshape(nB * nA, nCH, L)
    out = pl.pallas_call(
        _decode_body,
        grid=(nB * nA,),
        in_specs=[
            pl.BlockSpec(memory_space=pltpu.SMEM),
            pl.BlockSpec((1, nCH, L), lambda i: (i, 0, 0)),
        ],
        out_specs=pl.BlockSpec((1, L, nCH), lambda i: (i, 0, 0)),
        out_shape=jax.ShapeDtypeStruct((nB * nA, L, nCH), jnp.float32),
    )(scal, x)
    return out.reshape(nB, nA * L, nCH)
